# baseline (speedup 1.0000x reference)
import jax
import jax.numpy as jnp
from jax import lax
from jax.experimental import pallas as pl
from jax.experimental.pallas import tpu as pltpu
from jax.experimental.pallas import tpu_sc as plsc

BS, SEQ, D, PLEN, SESSIONS = 256, 196, 768, 10, 10
OUT_SEQ = PLEN + SEQ
XROW = SEQ * D
PROW = PLEN * D
OROW = OUT_SEQ * D
NC, NS = 2, 16
NW = NC * NS
SPW = BS // NW
NCHUNK = 12
CH = XROW // NCHUNK
NBUF = 4
TOT = SPW * NCHUNK


def _sc_body(x_hbm, pool_hbm, tid_hbm, out_hbm, idx_v, pv, bufs,
             sem_g, sem_in, sem_out):
    wid = lax.axis_index("s") * NC + lax.axis_index("c")
    base = pl.multiple_of(wid * SPW, SPW)

    pltpu.sync_copy(tid_hbm.at[pl.ds(base, SPW)], idx_v)
    gather = pltpu.make_async_copy(pool_hbm.at[idx_v], pv, sem_g)
    gather.start()

    def in_copy(c, b):
        j, p = c // NCHUNK, c % NCHUNK
        off = pl.multiple_of((base + j) * XROW + p * CH, 128)
        return pltpu.make_async_copy(
            x_hbm.at[pl.ds(off, CH)], bufs.at[b], sem_in.at[b])

    def out_copy(c, b):
        j, p = c // NCHUNK, c % NCHUNK
        off = pl.multiple_of((base + j) * OROW + PROW + p * CH, 128)
        return pltpu.make_async_copy(
            bufs.at[b], out_hbm.at[pl.ds(off, CH)], sem_out.at[b])

    in_copy(0, 0).start()
    in_copy(1, 1).start()
    for b in range(2):
        in_copy(b, b).wait()
        out_copy(b, b).start()
        in_copy(b + 2, b + 2).start()
    for b in range(2, NBUF):
        c = b
        in_copy(c, b).wait()
        out_copy(c, b).start()
        out_copy(c - 2, b - 2).wait()
        in_copy(c + 2, b - 2).start()

    def group(g, carry):
        for b in range(NBUF):
            c = g * NBUF + b
            in_copy(c, b).wait()
            out_copy(c, b).start()
            out_copy(c - 2, (b + 2) % NBUF).wait()
            in_copy(c + 2, (b + 2) % NBUF).start()
        return carry

    lax.fori_loop(1, TOT // NBUF - 1, group, 0)
    for b in range(NBUF):
        c = TOT - NBUF + b
        in_copy(c, b).wait()
        out_copy(c, b).start()
        out_copy(c - 2, (b + 2) % NBUF).wait()
        if c + 2 < TOT:
            in_copy(c + 2, (b + 2) % NBUF).start()
    out_copy(TOT - 2, (TOT - 2) % NBUF).wait()
    out_copy(TOT - 1, (TOT - 1) % NBUF).wait()

    gather.wait()
    for j in range(SPW):
        off = pl.multiple_of((base + j) * OROW, 128)
        pltpu.sync_copy(pv.at[j], out_hbm.at[pl.ds(off, PROW)])


def kernel(x, prompt_pool, task_id):
    mesh = plsc.VectorSubcoreMesh(core_axis_name="c", subcore_axis_name="s")
    run = pl.kernel(
        _sc_body,
        out_type=jax.ShapeDtypeStruct((BS * OROW,), jnp.float32),
        mesh=mesh,
        scratch_types=[
            pltpu.VMEM((SPW,), jnp.int32),
            pltpu.VMEM((SPW, PROW), jnp.float32),
            pltpu.VMEM((NBUF, CH), jnp.float32),
            pltpu.SemaphoreType.DMA,
            pltpu.SemaphoreType.DMA((NBUF,)),
            pltpu.SemaphoreType.DMA((NBUF,)),
        ],
    )
    out_flat = run(x.reshape(-1), prompt_pool.reshape(SESSIONS, PROW),
                   task_id.astype(jnp.int32))
    return out_flat.reshape(BS, OUT_SEQ, D)
